# baseline (speedup 1.0000x reference)
import functools

import jax
import jax.numpy as jnp
from jax.experimental import pallas as pl
from jax.experimental.pallas import tpu as pltpu

B = 24576
D = 100
EG = 4
EM = 4
NE = EG + EM + 1
NEDGE = B // 3


def _fused_kernel(mf0, mf1, mf2, spa_ref, rec_ref,
                  nf0, nf1, nf2, dg0, dg1, dg2, wg_ref,
                  c0_ref, c1_ref, srcw_ref, srcb_ref, dstw_ref, dstb_ref,
                  outw_ref, outb_ref, pos_ref, neg_ref, loss_ref, acc_ref,
                  *, br, nb):
    r3 = 3 * br
    wg = wg_ref[...]
    c0 = c0_ref[...]
    c1 = c1_ref[...]

    dot = lambda a, w: jax.lax.dot(a, w, preferred_element_type=jnp.float32)
    ones_row = jnp.ones((1, D), jnp.float32)
    bcast = lambda col: dot(col, ones_row)
    dot_hi = lambda a, w: jax.lax.dot(a, w,
                                      precision=jax.lax.Precision.HIGHEST,
                                      preferred_element_type=jnp.float32)

    mem = jnp.maximum(
        jnp.concatenate([mf0[...], mf1[...], mf2[...]], axis=0), 0.0)
    rec400 = jnp.maximum(rec_ref[...].reshape(r3, EM * D), 0.0)
    spa400 = spa_ref[...].reshape(r3, EG * D)
    s_e = [spa400[:, e * D:(e + 1) * D] for e in range(EG)]
    r_e = [rec400[:, e * D:(e + 1) * D] for e in range(EM)]
    nfs = jnp.concatenate([nf0[...], nf1[...], nf2[...]], axis=0)

    sub = jax.lax.broadcasted_iota(jnp.int32, (br, 128), 0) % 128
    lane = jax.lax.broadcasted_iota(jnp.int32, (br, 128), 1)
    diag = (sub == lane).astype(jnp.float32)
    ones128 = jnp.ones((128, D), jnp.float32)

    def expand_deg(dg):
        lt = jnp.log(dg[...].reshape(br // 128, 128)
                     .astype(jnp.float32) + 1.0)
        l512 = jnp.broadcast_to(lt[:, None, :],
                                (br // 128, 128, 128)).reshape(br, 128)
        return dot_hi(l512 * diag, ones128)

    ldb = jnp.concatenate(
        [expand_deg(dg0), expand_deg(dg1), expand_deg(dg2)], axis=0)
    scale = c0 + ldb * c1
    ms = (s_e[0] + s_e[1] + s_e[2] + s_e[3]) * (1.0 / EG)
    rspa = ms * scale
    rrec = (r_e[0] + r_e[1] + r_e[2] + r_e[3]) * (1.0 / EM)
    x1 = mem + rspa + rrec
    x2 = mem * rspa * rrec

    logits = (dot(mem, wg[0:D]) + dot(rspa, wg[D:2 * D])
              + dot(rrec, wg[2 * D:3 * D]) + dot(x1, wg[3 * D:4 * D])
              + dot(x2, wg[4 * D:5 * D]) + dot(nfs, wg[5 * D:6 * D]))

    idx = jax.lax.broadcasted_iota(jnp.int32, logits.shape, 1)
    m1 = jnp.max(logits, axis=1, keepdims=True)
    i1 = jnp.min(jnp.where(logits == m1, idx, NE), axis=1, keepdims=True)
    masked = jnp.where(idx == i1, -jnp.inf, logits)
    m2 = jnp.max(masked, axis=1, keepdims=True)
    i2 = jnp.min(jnp.where(masked == m2, idx, NE), axis=1, keepdims=True)
    e2 = jnp.exp(m2 - m1)
    denom = 1.0 + e2
    gates = (jnp.where(idx == i1, 1.0, 0.0)
             + jnp.where(idx == i2, e2, 0.0)) / denom

    gs = bcast(gates[:, 1:2]) * s_e[0]
    for e in range(1, EG):
        gs += bcast(gates[:, 1 + e:2 + e]) * s_e[e]
    rs = bcast(gates[:, 1 + EG:2 + EG]) * r_e[0]
    for e in range(1, EM):
        rs += bcast(gates[:, 1 + EG + e:2 + EG + e]) * r_e[e]
    out = bcast(gates[:, 0:1]) * mem + scale * gs + rs

    imp = jnp.sum(gates, axis=0, keepdims=True)
    ld_cnt = jnp.sum((gates > 0).astype(jnp.float32), axis=0, keepdims=True)
    i = pl.program_id(0)
    prev = jnp.where(i == 0, 0.0, acc_ref[...])
    acc_ref[...] = prev + jnp.concatenate([imp, ld_cnt], axis=0)

    h_src = dot(out[0:br], srcw_ref[...]) + srcb_ref[...]
    h_dst = dot(out[br:r3], dstw_ref[...]) + dstb_ref[...]
    h_pos = jnp.maximum(h_src + h_dst[0:br], 0.0)
    h_neg = jnp.maximum(h_src + h_dst[br:2 * br], 0.0)
    outw = outw_ref[...]
    ob = outb_ref[0, 0]
    pos_ref[...] = jnp.sum(h_pos * outw, axis=1, keepdims=True) + ob
    neg_ref[...] = jnp.sum(h_neg * outw, axis=1, keepdims=True) + ob

    @pl.when(i == nb - 1)
    def _():
        def cv2(x):
            mean = jnp.sum(x) / NE
            var = jnp.sum((x - mean) ** 2) / (NE - 1)
            return var / (mean * mean + 1e-10)
        loss_ref[...] = jnp.reshape(
            0.4 * (cv2(acc_ref[0:1, :]) + cv2(acc_ref[1:2, :])), (1, 1))


def kernel(memory_feats, spatial_out, recent_out, node_feats_src,
           node_degree, w_gate, deg_coef, src_W, src_b, dst_W, dst_b,
           out_W, out_b):
    BR = 512
    nb = NEDGE // BR

    spa2 = spatial_out.reshape(3, NEDGE, EG * D)
    rec2 = recent_out.reshape(3, NEDGE, EM * D)
    deg2 = node_degree.reshape(B // 512, 4, 128)
    c0 = deg_coef[0, :, 0].reshape(1, D)
    c1 = deg_coef[0, :, 1].reshape(1, D)
    srcb = src_b.reshape(1, D)
    dstb = dst_b.reshape(1, D)
    outw = out_W.reshape(1, D)
    outb = out_b.reshape(1, 1)

    body = functools.partial(_fused_kernel, br=BR, nb=nb)

    def rowspec(t):
        return pl.BlockSpec((BR, D), lambda i, t=t: (i + t * nb, 0))

    expspec = pl.BlockSpec((3, BR, EG * D), lambda i: (0, i, 0))

    def degspec(t):
        return pl.BlockSpec((BR // 512, 4, 128),
                            lambda i, t=t: (i + t * nb, 0, 0))

    full = lambda a: pl.BlockSpec(a.shape, lambda i: (0,) * a.ndim)

    pos, neg, loss = pl.pallas_call(
        body,
        grid=(nb,),
        in_specs=[
            rowspec(0), rowspec(1), rowspec(2),
            expspec, expspec,
            rowspec(0), rowspec(1), rowspec(2),
            degspec(0), degspec(1), degspec(2),
            full(w_gate),
            full(c0), full(c1),
            full(src_W), full(srcb),
            full(dst_W), full(dstb),
            full(outw), full(outb),
        ],
        out_specs=[
            pl.BlockSpec((BR, 1), lambda i: (i, 0)),
            pl.BlockSpec((BR, 1), lambda i: (i, 0)),
            pl.BlockSpec((1, 1), lambda i: (0, 0)),
        ],
        out_shape=[
            jax.ShapeDtypeStruct((NEDGE, 1), jnp.float32),
            jax.ShapeDtypeStruct((NEDGE, 1), jnp.float32),
            jax.ShapeDtypeStruct((1, 1), jnp.float32),
        ],
        scratch_shapes=[pltpu.VMEM((2, NE), jnp.float32)],
    )(memory_feats, memory_feats, memory_feats,
      spa2, rec2,
      node_feats_src, node_feats_src, node_feats_src,
      deg2, deg2, deg2,
      w_gate, c0, c1, src_W, srcb, dst_W, dstb, outw, outb)

    return (pos, neg, loss[0, 0])


# speedup vs baseline: 1.2304x; 1.2304x over previous
import functools

import jax
import jax.numpy as jnp
from jax.experimental import pallas as pl
from jax.experimental.pallas import tpu as pltpu

B = 24576
D = 100
EG = 4
EM = 4
NE = EG + EM + 1
NEDGE = B // 3


def _fused_kernel(mf_ref, spa_ref, rec_ref, nfs_ref, dg0, dg1, dg2, wg_ref,
                  c0_ref, c1_ref, srcw_ref, srcb_ref, dstw_ref, dstb_ref,
                  outw_ref, outb_ref, pos_ref, neg_ref, loss_ref, acc_ref,
                  *, br, nb):
    r3 = 3 * br
    wg = wg_ref[...]
    c0 = c0_ref[...]
    c1 = c1_ref[...]

    dot = lambda a, w: jax.lax.dot(a, w, preferred_element_type=jnp.float32)
    ones_row = jnp.ones((1, D), jnp.float32)
    bcast = lambda col: dot(col, ones_row)
    dot_hi = lambda a, w: jax.lax.dot(a, w,
                                      precision=jax.lax.Precision.HIGHEST,
                                      preferred_element_type=jnp.float32)

    mem = jnp.maximum(mf_ref[...].reshape(r3, D), 0.0)
    rec400 = jnp.maximum(rec_ref[...].reshape(r3, EM * D), 0.0)
    spa400 = spa_ref[...].reshape(r3, EG * D)
    s_e = [spa400[:, e * D:(e + 1) * D] for e in range(EG)]
    r_e = [rec400[:, e * D:(e + 1) * D] for e in range(EM)]
    nfs = nfs_ref[...].reshape(r3, D)

    sub = jax.lax.broadcasted_iota(jnp.int32, (br, 128), 0) % 128
    lane = jax.lax.broadcasted_iota(jnp.int32, (br, 128), 1)
    diag = (sub == lane).astype(jnp.float32)
    ones128 = jnp.ones((128, D), jnp.float32)

    def expand_deg(dg):
        lt = jnp.log(dg[...].reshape(br // 128, 128)
                     .astype(jnp.float32) + 1.0)
        l512 = jnp.broadcast_to(lt[:, None, :],
                                (br // 128, 128, 128)).reshape(br, 128)
        return dot_hi(l512 * diag, ones128)

    ldb = jnp.concatenate(
        [expand_deg(dg0), expand_deg(dg1), expand_deg(dg2)], axis=0)
    scale = c0 + ldb * c1
    ms = (s_e[0] + s_e[1] + s_e[2] + s_e[3]) * (1.0 / EG)
    rspa = ms * scale
    rrec = (r_e[0] + r_e[1] + r_e[2] + r_e[3]) * (1.0 / EM)
    x1 = mem + rspa + rrec
    x2 = mem * rspa * rrec

    logits = (dot(mem, wg[0:D]) + dot(rspa, wg[D:2 * D])
              + dot(rrec, wg[2 * D:3 * D]) + dot(x1, wg[3 * D:4 * D])
              + dot(x2, wg[4 * D:5 * D]) + dot(nfs, wg[5 * D:6 * D]))

    idx = jax.lax.broadcasted_iota(jnp.int32, logits.shape, 1)
    m1 = jnp.max(logits, axis=1, keepdims=True)
    i1 = jnp.min(jnp.where(logits == m1, idx, NE), axis=1, keepdims=True)
    masked = jnp.where(idx == i1, -jnp.inf, logits)
    m2 = jnp.max(masked, axis=1, keepdims=True)
    i2 = jnp.min(jnp.where(masked == m2, idx, NE), axis=1, keepdims=True)
    e2 = jnp.exp(m2 - m1)
    denom = 1.0 + e2
    gates = (jnp.where(idx == i1, 1.0, 0.0)
             + jnp.where(idx == i2, e2, 0.0)) / denom

    gs = bcast(gates[:, 1:2]) * s_e[0]
    for e in range(1, EG):
        gs += bcast(gates[:, 1 + e:2 + e]) * s_e[e]
    rs = bcast(gates[:, 1 + EG:2 + EG]) * r_e[0]
    for e in range(1, EM):
        rs += bcast(gates[:, 1 + EG + e:2 + EG + e]) * r_e[e]
    out = bcast(gates[:, 0:1]) * mem + scale * gs + rs

    imp = jnp.sum(gates, axis=0, keepdims=True)
    ld_cnt = jnp.sum((gates > 0).astype(jnp.float32), axis=0, keepdims=True)
    i = pl.program_id(0)
    prev = jnp.where(i == 0, 0.0, acc_ref[...])
    acc_ref[...] = prev + jnp.concatenate([imp, ld_cnt], axis=0)

    h_src = dot(out[0:br], srcw_ref[...]) + srcb_ref[...]
    h_dst = dot(out[br:r3], dstw_ref[...]) + dstb_ref[...]
    h_pos = jnp.maximum(h_src + h_dst[0:br], 0.0)
    h_neg = jnp.maximum(h_src + h_dst[br:2 * br], 0.0)
    outw = outw_ref[...]
    ob = outb_ref[0, 0]
    pos_ref[...] = jnp.sum(h_pos * outw, axis=1, keepdims=True) + ob
    neg_ref[...] = jnp.sum(h_neg * outw, axis=1, keepdims=True) + ob

    @pl.when(i == nb - 1)
    def _():
        def cv2(x):
            mean = jnp.sum(x) / NE
            var = jnp.sum((x - mean) ** 2) / (NE - 1)
            return var / (mean * mean + 1e-10)
        loss_ref[...] = jnp.reshape(
            0.4 * (cv2(acc_ref[0:1, :]) + cv2(acc_ref[1:2, :])), (1, 1))


def kernel(memory_feats, spatial_out, recent_out, node_feats_src,
           node_degree, w_gate, deg_coef, src_W, src_b, dst_W, dst_b,
           out_W, out_b):
    BR = 512
    nb = NEDGE // BR

    mf = memory_feats.reshape(3, NEDGE, D)
    spa = spatial_out.reshape(3, NEDGE, EG, D)
    rec = recent_out.reshape(3, NEDGE, EM, D)
    nfs = node_feats_src.reshape(3, NEDGE, D)
    deg2 = node_degree.reshape(B // 512, 4, 128)
    c0 = deg_coef[0, :, 0].reshape(1, D)
    c1 = deg_coef[0, :, 1].reshape(1, D)
    srcb = src_b.reshape(1, D)
    dstb = dst_b.reshape(1, D)
    outw = out_W.reshape(1, D)
    outb = out_b.reshape(1, 1)

    body = functools.partial(_fused_kernel, br=BR, nb=nb)

    rowspec = pl.BlockSpec((3, BR, D), lambda i: (0, i, 0))
    expspec = pl.BlockSpec((3, BR, EG, D), lambda i: (0, i, 0, 0))

    def degspec(t):
        return pl.BlockSpec((BR // 512, 4, 128),
                            lambda i, t=t: (i + t * nb, 0, 0))

    full = lambda a: pl.BlockSpec(a.shape, lambda i: (0,) * a.ndim)

    pos, neg, loss = pl.pallas_call(
        body,
        grid=(nb,),
        in_specs=[
            rowspec,
            expspec,
            expspec,
            rowspec,
            degspec(0), degspec(1), degspec(2),
            full(w_gate),
            full(c0), full(c1),
            full(src_W), full(srcb),
            full(dst_W), full(dstb),
            full(outw), full(outb),
        ],
        out_specs=[
            pl.BlockSpec((BR, 1), lambda i: (i, 0)),
            pl.BlockSpec((BR, 1), lambda i: (i, 0)),
            pl.BlockSpec((1, 1), lambda i: (0, 0)),
        ],
        out_shape=[
            jax.ShapeDtypeStruct((NEDGE, 1), jnp.float32),
            jax.ShapeDtypeStruct((NEDGE, 1), jnp.float32),
            jax.ShapeDtypeStruct((1, 1), jnp.float32),
        ],
        scratch_shapes=[pltpu.VMEM((2, NE), jnp.float32)],
    )(mf, spa, rec, nfs, deg2, deg2, deg2,
      w_gate, c0, c1, src_W, srcb, dst_W, dstb, outw, outb)

    return (pos, neg, loss[0, 0])


# speedup vs baseline: 1.2398x; 1.0076x over previous
import functools

import jax
import jax.numpy as jnp
from jax.experimental import pallas as pl
from jax.experimental.pallas import tpu as pltpu

B = 24576
D = 100
EG = 4
EM = 4
NE = EG + EM + 1
NEDGE = B // 3


def _fused_kernel(mf_ref, spa_ref, rec_ref, nfs_ref, dg0, dg1, dg2, wg_ref,
                  c0_ref, c1_ref, srcw_ref, srcb_ref, dstw_ref, dstb_ref,
                  outw_ref, outb_ref, pos_ref, neg_ref, loss_ref, acc_ref,
                  *, br, nb):
    r3 = 3 * br
    wg = wg_ref[...]
    c0 = c0_ref[...]
    c1 = c1_ref[...]

    dot = lambda a, w: jax.lax.dot(a, w, preferred_element_type=jnp.float32)
    ones_row = jnp.ones((1, D), jnp.float32)
    bcast = lambda col: dot(col, ones_row)
    dot_hi = lambda a, w: jax.lax.dot(a, w,
                                      precision=jax.lax.Precision.HIGHEST,
                                      preferred_element_type=jnp.float32)

    mem = jnp.maximum(mf_ref[...].reshape(r3, D), 0.0)
    rec400 = jnp.maximum(rec_ref[...].reshape(r3, EM * D), 0.0)
    spa400 = spa_ref[...].reshape(r3, EG * D)
    s_e = [spa400[:, e * D:(e + 1) * D] for e in range(EG)]
    r_e = [rec400[:, e * D:(e + 1) * D] for e in range(EM)]
    nfs = nfs_ref[...].reshape(r3, D)

    sub = jax.lax.broadcasted_iota(jnp.int32, (br, 128), 0) % 128
    lane = jax.lax.broadcasted_iota(jnp.int32, (br, 128), 1)
    diag = (sub == lane).astype(jnp.float32)
    ones128 = jnp.ones((128, D), jnp.float32)

    def expand_deg(dg):
        lt = jnp.log(dg[...].reshape(br // 128, 128)
                     .astype(jnp.float32) + 1.0)
        l512 = jnp.broadcast_to(lt[:, None, :],
                                (br // 128, 128, 128)).reshape(br, 128)
        return dot_hi(l512 * diag, ones128)

    ldb = jnp.concatenate(
        [expand_deg(dg0), expand_deg(dg1), expand_deg(dg2)], axis=0)
    scale = c0 + ldb * c1
    ms = (s_e[0] + s_e[1] + s_e[2] + s_e[3]) * (1.0 / EG)
    rspa = ms * scale
    rrec = (r_e[0] + r_e[1] + r_e[2] + r_e[3]) * (1.0 / EM)
    x1 = mem + rspa + rrec
    x2 = mem * rspa * rrec

    logits = (dot(mem, wg[0:D]) + dot(rspa, wg[D:2 * D])
              + dot(rrec, wg[2 * D:3 * D]) + dot(x1, wg[3 * D:4 * D])
              + dot(x2, wg[4 * D:5 * D]) + dot(nfs, wg[5 * D:6 * D]))

    idx = jax.lax.broadcasted_iota(jnp.int32, logits.shape, 1)
    m1 = jnp.max(logits, axis=1, keepdims=True)
    i1 = jnp.min(jnp.where(logits == m1, idx, NE), axis=1, keepdims=True)
    masked = jnp.where(idx == i1, -jnp.inf, logits)
    m2 = jnp.max(masked, axis=1, keepdims=True)
    i2 = jnp.min(jnp.where(masked == m2, idx, NE), axis=1, keepdims=True)
    e2 = jnp.exp(m2 - m1)
    denom = 1.0 + e2
    gates = (jnp.where(idx == i1, 1.0, 0.0)
             + jnp.where(idx == i2, e2, 0.0)) / denom

    gs = bcast(gates[:, 1:2]) * s_e[0]
    for e in range(1, EG):
        gs += bcast(gates[:, 1 + e:2 + e]) * s_e[e]
    rs = bcast(gates[:, 1 + EG:2 + EG]) * r_e[0]
    for e in range(1, EM):
        rs += bcast(gates[:, 1 + EG + e:2 + EG + e]) * r_e[e]
    out = bcast(gates[:, 0:1]) * mem + scale * gs + rs

    imp = jnp.sum(gates, axis=0, keepdims=True)
    ld_cnt = jnp.sum((gates > 0).astype(jnp.float32), axis=0, keepdims=True)
    i = pl.program_id(0)
    prev = jnp.where(i == 0, 0.0, acc_ref[...])
    acc_ref[...] = prev + jnp.concatenate([imp, ld_cnt], axis=0)

    h_src = dot(out[0:br], srcw_ref[...]) + srcb_ref[...]
    h_dst = dot(out[br:r3], dstw_ref[...]) + dstb_ref[...]
    h_pos = jnp.maximum(h_src + h_dst[0:br], 0.0)
    h_neg = jnp.maximum(h_src + h_dst[br:2 * br], 0.0)
    outw = outw_ref[...]
    ob = outb_ref[0, 0]
    pos_ref[...] = jnp.sum(h_pos * outw, axis=1, keepdims=True) + ob
    neg_ref[...] = jnp.sum(h_neg * outw, axis=1, keepdims=True) + ob

    @pl.when(i == nb - 1)
    def _():
        def cv2(x):
            mean = jnp.sum(x) / NE
            var = jnp.sum((x - mean) ** 2) / (NE - 1)
            return var / (mean * mean + 1e-10)
        loss_ref[...] = jnp.reshape(
            0.4 * (cv2(acc_ref[0:1, :]) + cv2(acc_ref[1:2, :])), (1, 1))


def kernel(memory_feats, spatial_out, recent_out, node_feats_src,
           node_degree, w_gate, deg_coef, src_W, src_b, dst_W, dst_b,
           out_W, out_b):
    BR = 1024
    nb = NEDGE // BR

    mf = memory_feats.reshape(3, NEDGE, D)
    spa = spatial_out.reshape(3, NEDGE, EG, D)
    rec = recent_out.reshape(3, NEDGE, EM, D)
    nfs = node_feats_src.reshape(3, NEDGE, D)
    deg2 = node_degree.reshape(B // 512, 4, 128)
    c0 = deg_coef[0, :, 0].reshape(1, D)
    c1 = deg_coef[0, :, 1].reshape(1, D)
    srcb = src_b.reshape(1, D)
    dstb = dst_b.reshape(1, D)
    outw = out_W.reshape(1, D)
    outb = out_b.reshape(1, 1)

    body = functools.partial(_fused_kernel, br=BR, nb=nb)

    rowspec = pl.BlockSpec((3, BR, D), lambda i: (0, i, 0))
    expspec = pl.BlockSpec((3, BR, EG, D), lambda i: (0, i, 0, 0))

    def degspec(t):
        return pl.BlockSpec((BR // 512, 4, 128),
                            lambda i, t=t: (i + t * nb, 0, 0))

    full = lambda a: pl.BlockSpec(a.shape, lambda i: (0,) * a.ndim)

    pos, neg, loss = pl.pallas_call(
        body,
        grid=(nb,),
        in_specs=[
            rowspec,
            expspec,
            expspec,
            rowspec,
            degspec(0), degspec(1), degspec(2),
            full(w_gate),
            full(c0), full(c1),
            full(src_W), full(srcb),
            full(dst_W), full(dstb),
            full(outw), full(outb),
        ],
        out_specs=[
            pl.BlockSpec((BR, 1), lambda i: (i, 0)),
            pl.BlockSpec((BR, 1), lambda i: (i, 0)),
            pl.BlockSpec((1, 1), lambda i: (0, 0)),
        ],
        out_shape=[
            jax.ShapeDtypeStruct((NEDGE, 1), jnp.float32),
            jax.ShapeDtypeStruct((NEDGE, 1), jnp.float32),
            jax.ShapeDtypeStruct((1, 1), jnp.float32),
        ],
        scratch_shapes=[pltpu.VMEM((2, NE), jnp.float32)],
    )(mf, spa, rec, nfs, deg2, deg2, deg2,
      w_gate, c0, c1, src_W, srcb, dst_W, dstb, outw, outb)

    return (pos, neg, loss[0, 0])


# speedup vs baseline: 1.3078x; 1.0549x over previous
import functools

import jax
import jax.numpy as jnp
from jax.experimental import pallas as pl
from jax.experimental.pallas import tpu as pltpu

B = 24576
D = 100
EG = 4
EM = 4
NE = EG + EM + 1
NEDGE = B // 3


def _fused_kernel(mf0, mf1, mf2, spa_ref, rec_ref, nf0, nf1, nf2,
                  dg0, dg1, dg2, wg_ref,
                  c0_ref, c1_ref, srcw_ref, srcb_ref, dstw_ref, dstb_ref,
                  outw_ref, outb_ref, pos_ref, neg_ref, loss_ref, acc_ref,
                  *, br, nb):
    r3 = 3 * br
    wg = wg_ref[...]
    c0 = c0_ref[...]
    c1 = c1_ref[...]

    dot = lambda a, w: jax.lax.dot(a, w, preferred_element_type=jnp.float32)
    ones_row = jnp.ones((1, D), jnp.float32)
    bcast = lambda col: dot(col, ones_row)
    dot_hi = lambda a, w: jax.lax.dot(a, w,
                                      precision=jax.lax.Precision.HIGHEST,
                                      preferred_element_type=jnp.float32)

    mem = jnp.maximum(
        jnp.concatenate([mf0[...], mf1[...], mf2[...]], axis=0), 0.0)
    rec400 = jnp.maximum(rec_ref[...].reshape(r3, EM * D), 0.0)
    spa400 = spa_ref[...].reshape(r3, EG * D)
    s_e = [spa400[:, e * D:(e + 1) * D] for e in range(EG)]
    r_e = [rec400[:, e * D:(e + 1) * D] for e in range(EM)]
    nfs = jnp.concatenate([nf0[...], nf1[...], nf2[...]], axis=0)

    sub = jax.lax.broadcasted_iota(jnp.int32, (br, 128), 0) % 128
    lane = jax.lax.broadcasted_iota(jnp.int32, (br, 128), 1)
    diag = (sub == lane).astype(jnp.float32)
    ones128 = jnp.ones((128, D), jnp.float32)

    def expand_deg(dg):
        lt = jnp.log(dg[...].reshape(br // 128, 128)
                     .astype(jnp.float32) + 1.0)
        l512 = jnp.broadcast_to(lt[:, None, :],
                                (br // 128, 128, 128)).reshape(br, 128)
        return dot_hi(l512 * diag, ones128)

    ldb = jnp.concatenate(
        [expand_deg(dg0), expand_deg(dg1), expand_deg(dg2)], axis=0)
    scale = c0 + ldb * c1
    ms = (s_e[0] + s_e[1] + s_e[2] + s_e[3]) * (1.0 / EG)
    rspa = ms * scale
    rrec = (r_e[0] + r_e[1] + r_e[2] + r_e[3]) * (1.0 / EM)
    x1 = mem + rspa + rrec
    x2 = mem * rspa * rrec

    logits = (dot(mem, wg[0:D]) + dot(rspa, wg[D:2 * D])
              + dot(rrec, wg[2 * D:3 * D]) + dot(x1, wg[3 * D:4 * D])
              + dot(x2, wg[4 * D:5 * D]) + dot(nfs, wg[5 * D:6 * D]))

    idx = jax.lax.broadcasted_iota(jnp.int32, logits.shape, 1)
    m1 = jnp.max(logits, axis=1, keepdims=True)
    i1 = jnp.min(jnp.where(logits == m1, idx, NE), axis=1, keepdims=True)
    masked = jnp.where(idx == i1, -jnp.inf, logits)
    m2 = jnp.max(masked, axis=1, keepdims=True)
    i2 = jnp.min(jnp.where(masked == m2, idx, NE), axis=1, keepdims=True)
    e2 = jnp.exp(m2 - m1)
    denom = 1.0 + e2
    gates = (jnp.where(idx == i1, 1.0, 0.0)
             + jnp.where(idx == i2, e2, 0.0)) / denom

    gs = bcast(gates[:, 1:2]) * s_e[0]
    for e in range(1, EG):
        gs += bcast(gates[:, 1 + e:2 + e]) * s_e[e]
    rs = bcast(gates[:, 1 + EG:2 + EG]) * r_e[0]
    for e in range(1, EM):
        rs += bcast(gates[:, 1 + EG + e:2 + EG + e]) * r_e[e]
    out = bcast(gates[:, 0:1]) * mem + scale * gs + rs

    imp = jnp.sum(gates, axis=0, keepdims=True)
    ld_cnt = jnp.sum((gates > 0).astype(jnp.float32), axis=0, keepdims=True)
    i = pl.program_id(0)
    prev = jnp.where(i == 0, 0.0, acc_ref[...])
    acc_ref[...] = prev + jnp.concatenate([imp, ld_cnt], axis=0)

    h_src = dot(out[0:br], srcw_ref[...]) + srcb_ref[...]
    h_dst = dot(out[br:r3], dstw_ref[...]) + dstb_ref[...]
    h_pos = jnp.maximum(h_src + h_dst[0:br], 0.0)
    h_neg = jnp.maximum(h_src + h_dst[br:2 * br], 0.0)
    outw = outw_ref[...]
    ob = outb_ref[0, 0]
    pos_ref[...] = jnp.sum(h_pos * outw, axis=1, keepdims=True) + ob
    neg_ref[...] = jnp.sum(h_neg * outw, axis=1, keepdims=True) + ob

    @pl.when(i == nb - 1)
    def _():
        def cv2(x):
            mean = jnp.sum(x) / NE
            var = jnp.sum((x - mean) ** 2) / (NE - 1)
            return var / (mean * mean + 1e-10)
        loss_ref[...] = jnp.reshape(
            0.4 * (cv2(acc_ref[0:1, :]) + cv2(acc_ref[1:2, :])), (1, 1))


def kernel(memory_feats, spatial_out, recent_out, node_feats_src,
           node_degree, w_gate, deg_coef, src_W, src_b, dst_W, dst_b,
           out_W, out_b):
    BR = 1024
    nb = NEDGE // BR

    spa = spatial_out.reshape(3, NEDGE, EG, D)
    rec = recent_out.reshape(3, NEDGE, EM, D)
    deg2 = node_degree.reshape(B // 512, 4, 128)
    c0 = deg_coef[0, :, 0].reshape(1, D)
    c1 = deg_coef[0, :, 1].reshape(1, D)
    srcb = src_b.reshape(1, D)
    dstb = dst_b.reshape(1, D)
    outw = out_W.reshape(1, D)
    outb = out_b.reshape(1, 1)

    body = functools.partial(_fused_kernel, br=BR, nb=nb)

    def rowspec(t):
        return pl.BlockSpec((BR, D), lambda i, t=t: (i + t * nb, 0))

    expspec = pl.BlockSpec((3, BR, EG, D), lambda i: (0, i, 0, 0))

    def degspec(t):
        return pl.BlockSpec((BR // 512, 4, 128),
                            lambda i, t=t: (i + t * nb, 0, 0))

    full = lambda a: pl.BlockSpec(a.shape, lambda i: (0,) * a.ndim)

    pos, neg, loss = pl.pallas_call(
        body,
        grid=(nb,),
        in_specs=[
            rowspec(0), rowspec(1), rowspec(2),
            expspec,
            expspec,
            rowspec(0), rowspec(1), rowspec(2),
            degspec(0), degspec(1), degspec(2),
            full(w_gate),
            full(c0), full(c1),
            full(src_W), full(srcb),
            full(dst_W), full(dstb),
            full(outw), full(outb),
        ],
        out_specs=[
            pl.BlockSpec((BR, 1), lambda i: (i, 0)),
            pl.BlockSpec((BR, 1), lambda i: (i, 0)),
            pl.BlockSpec((1, 1), lambda i: (0, 0)),
        ],
        out_shape=[
            jax.ShapeDtypeStruct((NEDGE, 1), jnp.float32),
            jax.ShapeDtypeStruct((NEDGE, 1), jnp.float32),
            jax.ShapeDtypeStruct((1, 1), jnp.float32),
        ],
        scratch_shapes=[pltpu.VMEM((2, NE), jnp.float32)],
    )(memory_feats, memory_feats, memory_feats, spa, rec,
      node_feats_src, node_feats_src, node_feats_src, deg2, deg2, deg2,
      w_gate, c0, c1, src_W, srcb, dst_W, dstb, outw, outb)

    return (pos, neg, loss[0, 0])


# speedup vs baseline: 1.3199x; 1.0093x over previous
import functools

import jax
import jax.numpy as jnp
from jax.experimental import pallas as pl
from jax.experimental.pallas import tpu as pltpu

B = 24576
D = 100
EG = 4
EM = 4
NE = EG + EM + 1
NEDGE = B // 3


def _fused_kernel(mf0, mf1, mf2, spa_ref, rec_ref, nf0, nf1, nf2,
                  dg0, dg1, dg2, wg_ref,
                  c0_ref, c1_ref, srcw_ref, srcb_ref, dstw_ref, dstb_ref,
                  outw_ref, outb_ref, pos_ref, neg_ref, loss_ref, acc_ref,
                  *, br, nb):
    r3 = 3 * br
    wg = wg_ref[...]
    c0 = c0_ref[...]
    c1 = c1_ref[...]

    dot = lambda a, w: jax.lax.dot(a, w, preferred_element_type=jnp.float32)
    ones_row = jnp.ones((1, D), jnp.float32)
    bcast = lambda col: dot(col, ones_row)
    dot_hi = lambda a, w: jax.lax.dot(a, w,
                                      precision=jax.lax.Precision.HIGHEST,
                                      preferred_element_type=jnp.float32)

    mem = jnp.maximum(
        jnp.concatenate([mf0[...], mf1[...], mf2[...]], axis=0), 0.0)
    rec400 = jnp.maximum(rec_ref[...].reshape(r3, EM * D), 0.0)
    spa400 = spa_ref[...].reshape(r3, EG * D)
    s_e = [spa400[:, e * D:(e + 1) * D] for e in range(EG)]
    r_e = [rec400[:, e * D:(e + 1) * D] for e in range(EM)]
    nfs = jnp.concatenate([nf0[...], nf1[...], nf2[...]], axis=0)

    sub = jax.lax.broadcasted_iota(jnp.int32, (br, 128), 0) % 128
    lane = jax.lax.broadcasted_iota(jnp.int32, (br, 128), 1)
    diag = (sub == lane).astype(jnp.float32)
    ones128 = jnp.ones((128, D), jnp.float32)

    def expand_deg(dg):
        lt = jnp.log(dg[...].reshape(br // 128, 128)
                     .astype(jnp.float32) + 1.0)
        l512 = jnp.broadcast_to(lt[:, None, :],
                                (br // 128, 128, 128)).reshape(br, 128)
        return dot_hi(l512 * diag, ones128)

    ldb = jnp.concatenate(
        [expand_deg(dg0), expand_deg(dg1), expand_deg(dg2)], axis=0)
    scale = c0 + ldb * c1
    ms = (s_e[0] + s_e[1] + s_e[2] + s_e[3]) * (1.0 / EG)
    rspa = ms * scale
    rrec = (r_e[0] + r_e[1] + r_e[2] + r_e[3]) * (1.0 / EM)
    x1 = mem + rspa + rrec
    x2 = mem * rspa * rrec

    padl = lambda a: jnp.pad(a, ((0, 0), (0, 128 - D)))
    cat768 = jnp.concatenate(
        [padl(mem), padl(rspa), padl(rrec), padl(x1), padl(x2), padl(nfs)],
        axis=1)
    logits = dot(cat768, wg)

    idx = jax.lax.broadcasted_iota(jnp.int32, logits.shape, 1)
    m1 = jnp.max(logits, axis=1, keepdims=True)
    i1 = jnp.min(jnp.where(logits == m1, idx, NE), axis=1, keepdims=True)
    masked = jnp.where(idx == i1, -jnp.inf, logits)
    m2 = jnp.max(masked, axis=1, keepdims=True)
    i2 = jnp.min(jnp.where(masked == m2, idx, NE), axis=1, keepdims=True)
    e2 = jnp.exp(m2 - m1)
    denom = 1.0 + e2
    gates = (jnp.where(idx == i1, 1.0, 0.0)
             + jnp.where(idx == i2, e2, 0.0)) / denom

    gs = bcast(gates[:, 1:2]) * s_e[0]
    for e in range(1, EG):
        gs += bcast(gates[:, 1 + e:2 + e]) * s_e[e]
    rs = bcast(gates[:, 1 + EG:2 + EG]) * r_e[0]
    for e in range(1, EM):
        rs += bcast(gates[:, 1 + EG + e:2 + EG + e]) * r_e[e]
    out = bcast(gates[:, 0:1]) * mem + scale * gs + rs

    imp = jnp.sum(gates, axis=0, keepdims=True)
    ld_cnt = jnp.sum((gates > 0).astype(jnp.float32), axis=0, keepdims=True)
    i = pl.program_id(0)
    prev = jnp.where(i == 0, 0.0, acc_ref[...])
    acc_ref[...] = prev + jnp.concatenate([imp, ld_cnt], axis=0)

    h_src = dot(out[0:br], srcw_ref[...]) + srcb_ref[...]
    h_dst = dot(out[br:r3], dstw_ref[...]) + dstb_ref[...]
    h_pos = jnp.maximum(h_src + h_dst[0:br], 0.0)
    h_neg = jnp.maximum(h_src + h_dst[br:2 * br], 0.0)
    outw = outw_ref[...]
    ob = outb_ref[0, 0]
    pos_ref[...] = jnp.sum(h_pos * outw, axis=1, keepdims=True) + ob
    neg_ref[...] = jnp.sum(h_neg * outw, axis=1, keepdims=True) + ob

    @pl.when(i == nb - 1)
    def _():
        def cv2(x):
            mean = jnp.sum(x) / NE
            var = jnp.sum((x - mean) ** 2) / (NE - 1)
            return var / (mean * mean + 1e-10)
        loss_ref[...] = jnp.reshape(
            0.4 * (cv2(acc_ref[0:1, :]) + cv2(acc_ref[1:2, :])), (1, 1))


def kernel(memory_feats, spatial_out, recent_out, node_feats_src,
           node_degree, w_gate, deg_coef, src_W, src_b, dst_W, dst_b,
           out_W, out_b):
    BR = 1024
    nb = NEDGE // BR

    spa = spatial_out.reshape(3, NEDGE, EG, D)
    rec = recent_out.reshape(3, NEDGE, EM, D)
    deg2 = node_degree.reshape(B // 512, 4, 128)
    wgp = jnp.pad(w_gate.reshape(6, D, NE),
                  ((0, 0), (0, 128 - D), (0, 0))).reshape(6 * 128, NE)
    c0 = deg_coef[0, :, 0].reshape(1, D)
    c1 = deg_coef[0, :, 1].reshape(1, D)
    srcb = src_b.reshape(1, D)
    dstb = dst_b.reshape(1, D)
    outw = out_W.reshape(1, D)
    outb = out_b.reshape(1, 1)

    body = functools.partial(_fused_kernel, br=BR, nb=nb)

    def rowspec(t):
        return pl.BlockSpec((BR, D), lambda i, t=t: (i + t * nb, 0))

    expspec = pl.BlockSpec((3, BR, EG, D), lambda i: (0, i, 0, 0))

    def degspec(t):
        return pl.BlockSpec((BR // 512, 4, 128),
                            lambda i, t=t: (i + t * nb, 0, 0))

    full = lambda a: pl.BlockSpec(a.shape, lambda i: (0,) * a.ndim)

    pos, neg, loss = pl.pallas_call(
        body,
        grid=(nb,),
        in_specs=[
            rowspec(0), rowspec(1), rowspec(2),
            expspec,
            expspec,
            rowspec(0), rowspec(1), rowspec(2),
            degspec(0), degspec(1), degspec(2),
            full(wgp),
            full(c0), full(c1),
            full(src_W), full(srcb),
            full(dst_W), full(dstb),
            full(outw), full(outb),
        ],
        out_specs=[
            pl.BlockSpec((BR, 1), lambda i: (i, 0)),
            pl.BlockSpec((BR, 1), lambda i: (i, 0)),
            pl.BlockSpec((1, 1), lambda i: (0, 0)),
        ],
        out_shape=[
            jax.ShapeDtypeStruct((NEDGE, 1), jnp.float32),
            jax.ShapeDtypeStruct((NEDGE, 1), jnp.float32),
            jax.ShapeDtypeStruct((1, 1), jnp.float32),
        ],
        scratch_shapes=[pltpu.VMEM((2, NE), jnp.float32)],
    )(memory_feats, memory_feats, memory_feats, spa, rec,
      node_feats_src, node_feats_src, node_feats_src, deg2, deg2, deg2,
      wgp, c0, c1, src_W, srcb, dst_W, dstb, outw, outb)

    return (pos, neg, loss[0, 0])
